# Initial kernel scaffold; baseline (speedup 1.0000x reference)
import jax
import jax.numpy as jnp
from jax.experimental import pallas as pl


def kernel(x_content, edge_index, edge_type, W0, root0, b0, W1, root1, b1, Wout, bout):
    raise NotImplementedError("write your pallas kernel here")


# speedup vs baseline: 2.7616x; 2.7616x over previous
import functools

import jax
import jax.numpy as jnp
from jax import lax
from jax.experimental import pallas as pl
from jax.experimental.pallas import tpu as pltpu
from jax.experimental.pallas import tpu_sc as plsc

N = 10000
E = 320000
D = 128
R = 3
C = 4

NC = 2
NS = 16
NHALF = N // NC
NLOCP = 5120
T = R * NLOCP
EPT = E // NS
G = 32
NCHUNK = EPT // G
ZROWS = 8
TPT = T // NS


def _sc_body(src, dst, typ, x, acc_out, cnt_out, acc_s, cnt_s, sem):
    pl.run_scoped(
        functools.partial(_sc_tile, src, dst, typ, x, acc_out, cnt_out,
                          acc_s, cnt_s, sem),
        pltpu.VMEM((G,), jnp.int32),
        pltpu.VMEM((G,), jnp.int32),
        pltpu.VMEM((G,), jnp.int32),
        pltpu.VMEM((G,), jnp.int32),
        pltpu.VMEM((G, D), jnp.float32),
        pltpu.VMEM((G,), jnp.float32),
        pltpu.VMEM((ZROWS, D), jnp.float32),
        pltpu.VMEM((TPT,), jnp.float32),
    )


def _sc_tile(src, dst, typ, x, acc_out, cnt_out, acc_s, cnt_s, sem,
             gidx, dstc, typc, sidx, rows, wv, zrow, zcnt):
    c = lax.axis_index("c")
    s = lax.axis_index("s")
    nb = c * NHALF

    def zr(i, carry):
        zrow[i // 8, pl.ds((i % 8) * 16, 16)] = jnp.zeros((16,), jnp.float32)
        return carry
    lax.fori_loop(0, ZROWS * 8, zr, 0)

    def zc(i, carry):
        zcnt[pl.ds(i * 16, 16)] = jnp.zeros((16,), jnp.float32)
        return carry
    lax.fori_loop(0, TPT // 16, zc, 0)

    def zs(j, carry):
        pltpu.sync_copy(zrow, acc_s.at[pl.ds(s * TPT + j * ZROWS, ZROWS)])
        return carry
    lax.fori_loop(0, TPT // ZROWS, zs, 0)
    pltpu.sync_copy(zcnt, cnt_s.at[pl.ds(s * TPT, TPT)])
    plsc.subcore_barrier()

    def chunk(i, carry):
        base = s * EPT + i * G
        pltpu.sync_copy(src.at[pl.ds(base, G)], gidx)
        pltpu.sync_copy(dst.at[pl.ds(base, G)], dstc)
        pltpu.sync_copy(typ.at[pl.ds(base, G)], typc)
        for k in range(G // 16):
            d16 = dstc[pl.ds(k * 16, 16)]
            t16 = typc[pl.ds(k * 16, 16)]
            m = (d16 >= nb) & (d16 < nb + NHALF)
            loc = jnp.where(m, d16 - nb, NHALF + (d16 & 63))
            sidx[pl.ds(k * 16, 16)] = t16 * NLOCP + loc
            wv[pl.ds(k * 16, 16)] = jnp.where(m, jnp.float32(1.0), jnp.float32(0.0))
        pltpu.async_copy(x.at[gidx], rows, sem).wait()
        pltpu.sync_copy(rows, acc_s.at[sidx], add=True)
        pltpu.sync_copy(wv, cnt_s.at[sidx], add=True)
        return carry
    lax.fori_loop(0, NCHUNK, chunk, 0)

    plsc.subcore_barrier()
    def cpout(j, carry):
        pltpu.sync_copy(acc_s.at[pl.ds(s * TPT + j * ZROWS, ZROWS)], zrow)
        pltpu.sync_copy(zrow, acc_out.at[c, pl.ds(s * TPT + j * ZROWS, ZROWS)])
        return carry
    lax.fori_loop(0, TPT // ZROWS, cpout, 0)
    pltpu.sync_copy(cnt_s.at[pl.ds(s * TPT, TPT)], zcnt)
    pltpu.sync_copy(zcnt, cnt_out.at[pl.ds(c * T + s * TPT, TPT)])


_MESH = plsc.VectorSubcoreMesh(core_axis_name="c", subcore_axis_name="s")

_sc_scatter = functools.partial(
    pl.kernel,
    mesh=_MESH,
    out_type=[
        jax.ShapeDtypeStruct((NC, T, D), jnp.float32),
        jax.ShapeDtypeStruct((NC * T,), jnp.float32),
    ],
    scratch_types=[
        pltpu.VMEM_SHARED((T, D), jnp.float32) @ _MESH,
        pltpu.VMEM_SHARED((T,), jnp.float32) @ _MESH,
        pltpu.SemaphoreType.DMA @ _MESH,
    ],
)(_sc_body)


def _tc_body(x_ref, acc_ref, cnt_ref, W1_ref, root1_ref, b1_ref,
             Wout_ref, bout_ref, o_ref):
    xb = x_ref[...]
    h = jnp.dot(xb, root1_ref[...], preferred_element_type=jnp.float32)
    h = h + b1_ref[0]
    cnt = cnt_ref[0].reshape(T)
    for r in range(R):
        A = acc_ref[0, r * NLOCP:r * NLOCP + NHALF, :]
        cr = jnp.maximum(cnt[r * NLOCP:r * NLOCP + NHALF], 1.0)
        h = h + jnp.dot(A / cr[:, None], W1_ref[r],
                        preferred_element_type=jnp.float32)
    h = jnp.maximum(h, 0.0)
    o_ref[...] = jnp.dot(h, Wout_ref[...],
                         preferred_element_type=jnp.float32) + bout_ref[0]


def kernel(x_content, edge_index, edge_type, W0, root0, b0,
           W1, root1, b1, Wout, bout):
    src = edge_index[0]
    dst = edge_index[1]
    acc, cnt = _sc_scatter(src, dst, edge_type, x_content)
    cnt3 = cnt.reshape(NC, T // 128, 128)
    out = pl.pallas_call(
        _tc_body,
        grid=(NC,),
        in_specs=[
            pl.BlockSpec((NHALF, D), lambda c: (c, 0)),
            pl.BlockSpec((1, T, D), lambda c: (c, 0, 0)),
            pl.BlockSpec((1, T // 128, 128), lambda c: (c, 0, 0)),
            pl.BlockSpec((R, D, D), lambda c: (0, 0, 0)),
            pl.BlockSpec((D, D), lambda c: (0, 0)),
            pl.BlockSpec((1, D), lambda c: (0, 0)),
            pl.BlockSpec((D, C), lambda c: (0, 0)),
            pl.BlockSpec((1, C), lambda c: (0, 0)),
        ],
        out_specs=pl.BlockSpec((NHALF, C), lambda c: (c, 0)),
        out_shape=jax.ShapeDtypeStruct((N, C), jnp.float32),
    )(x_content, acc, cnt3, W1, root1, b1.reshape(1, D),
      Wout, bout.reshape(1, C))
    return out
